# baseline (speedup 1.0000x reference)
import functools

import jax
import jax.numpy as jnp
from jax import lax
from jax.experimental import pallas as pl
from jax.experimental.pallas import tpu as pltpu
from jax.experimental.pallas import tpu_sc as plsc

N = 10000
E = 320000
D = 128
D_OUT = 64

NC = 2
NS = 16
NW = NC * NS
CH = 128
NCHUNK = 80
GRP = 16
NCHUNKS_TOTAL = 2560
CN0 = 32
CN1 = 128
NGRP0 = CN0 // GRP
NGRP1 = CN1 // GRP
C1_BASE = NS * CN0
EP = NW * NCHUNK * CH
NPAD = 10016
RSTRIDE = 624
RSZ = 656
DEGW = 16

_sc_mesh = plsc.VectorSubcoreMesh(core_axis_name="c", subcore_axis_name="s")


@functools.partial(
    pl.kernel,
    out_type=jax.ShapeDtypeStruct((NC, NPAD, D), jnp.float32),
    mesh=_sc_mesh,
    scratch_types=[
        pltpu.VMEM((NCHUNK, CH), jnp.int32),
        pltpu.VMEM((CH, D), jnp.float32),
        pltpu.VMEM_SHARED((NPAD, D), jnp.float32),
    ],
)
def _deg_kernel(col_hbm, ones_hbm, zeros_hbm, deg_out, idx_v, ones_v, table_s):
    c = lax.axis_index("c")
    s = lax.axis_index("s")
    wid = c * NS + s
    sl = pl.ds(s * RSTRIDE, RSZ)
    pltpu.sync_copy(col_hbm.at[wid], idx_v)
    pltpu.sync_copy(ones_hbm, ones_v)
    pltpu.sync_copy(zeros_hbm, table_s.at[sl])
    plsc.subcore_barrier()

    def body(j, carry):
        pltpu.sync_copy(ones_v, table_s.at[idx_v.at[j]], add=True)
        return carry

    lax.fori_loop(0, NCHUNK, body, 0)
    plsc.subcore_barrier()
    pltpu.sync_copy(table_s.at[sl], deg_out.at[c, sl])


@functools.partial(
    pl.kernel,
    out_type=jax.ShapeDtypeStruct((NC, NPAD, D), jnp.float32),
    mesh=_sc_mesh,
    scratch_types=[
        pltpu.VMEM((GRP, CH), jnp.int32),
        pltpu.VMEM((GRP, CH), jnp.int32),
        pltpu.VMEM((CH, D), jnp.float32),
        pltpu.VMEM((CH, D), jnp.float32),
        pltpu.VMEM_SHARED((NPAD, D), jnp.float32),
        pltpu.SemaphoreType.DMA,
        pltpu.SemaphoreType.DMA,
    ],
)
def _gs_kernel(hp_hbm, row_hbm, col_hbm, zeros_hbm, acc_out,
               row_v, col_v, bufa, bufb, acc_s, sema, semb):
    c = lax.axis_index("c")
    s = lax.axis_index("s")
    sl = pl.ds(s * RSTRIDE, RSZ)
    base16 = jnp.where(c == 0, (CN0 // 16) * s, C1_BASE // 16 + (CN1 // 16) * s)
    ngrp = jnp.where(c == 0, NGRP0, NGRP1)

    def _stage(g):
        off = pl.ds(16 * (base16 + g * (GRP // 16)), GRP)
        pltpu.sync_copy(row_hbm.at[off], row_v)
        pltpu.sync_copy(col_hbm.at[off], col_v)
        pltpu.async_copy(hp_hbm.at[row_v.at[0]], bufa, sema)

    _stage(0)
    pltpu.sync_copy(zeros_hbm, acc_s.at[sl])
    plsc.subcore_barrier()

    def body(jj, carry2):
        j0 = jj * 2
        pltpu.async_copy(hp_hbm.at[row_v.at[j0 + 1]], bufb, semb)
        pltpu.make_async_copy(hp_hbm.at[row_v.at[j0]], bufa, sema).wait()
        pltpu.sync_copy(bufa, acc_s.at[col_v.at[j0]], add=True)

        @pl.when(jj < GRP // 2 - 1)
        def _():
            pltpu.async_copy(hp_hbm.at[row_v.at[j0 + 2]], bufa, sema)

        pltpu.make_async_copy(hp_hbm.at[row_v.at[j0 + 1]], bufb, semb).wait()
        pltpu.sync_copy(bufb, acc_s.at[col_v.at[j0 + 1]], add=True)
        return carry2

    def group(g, carry):
        lax.fori_loop(0, GRP // 2, body, 0)

        @pl.when(g + 1 < ngrp)
        def _():
            _stage(g + 1)

        return carry

    lax.fori_loop(0, ngrp, group, 0)
    plsc.subcore_barrier()
    pltpu.sync_copy(acc_s.at[sl], acc_out.at[c, sl])


BLK = 2000
GRID = N // BLK

_xwt = lambda a, b: lax.dot_general(
    a, b, (((1,), (1,)), ((), ())), preferred_element_type=jnp.float32)


def _mm1_body(deg_ref, x_ref, w_ref, dinv_ref, hp_ref):
    deg = deg_ref[0, :, 0:1] + deg_ref[1, :, 0:1] + 1.0
    dinv = lax.rsqrt(deg)
    dinv_ref[...] = dinv
    hp_ref[...] = _xwt(x_ref[...], w_ref[...]) * dinv


_mm1 = pl.pallas_call(
    _mm1_body,
    grid=(GRID,),
    in_specs=[
        pl.BlockSpec((NC, BLK, D), lambda i: (0, i, 0)),
        pl.BlockSpec((BLK, D), lambda i: (i, 0)),
        pl.BlockSpec((D, D), lambda i: (0, 0)),
    ],
    out_specs=[
        pl.BlockSpec((BLK, 1), lambda i: (i, 0)),
        pl.BlockSpec((BLK, D), lambda i: (i, 0)),
    ],
    out_shape=[
        jax.ShapeDtypeStruct((N, 1), jnp.float32),
        jax.ShapeDtypeStruct((N, D), jnp.float32),
    ],
)


def _mid_body(acc_ref, hprev_ref, dinv_ref, b_ref, w_ref, hp_ref):
    dinv = dinv_ref[...]
    pre = (acc_ref[0] + acc_ref[1] + hprev_ref[...]) * dinv + b_ref[...]
    xn = jnp.maximum(pre, 0.0)
    hp_ref[...] = _xwt(xn, w_ref[...]) * dinv


_mid = pl.pallas_call(
    _mid_body,
    grid=(GRID,),
    in_specs=[
        pl.BlockSpec((NC, BLK, D), lambda i: (0, i, 0)),
        pl.BlockSpec((BLK, D), lambda i: (i, 0)),
        pl.BlockSpec((BLK, 1), lambda i: (i, 0)),
        pl.BlockSpec((1, D), lambda i: (0, 0)),
        pl.BlockSpec((D, D), lambda i: (0, 0)),
    ],
    out_specs=pl.BlockSpec((BLK, D), lambda i: (i, 0)),
    out_shape=jax.ShapeDtypeStruct((N, D), jnp.float32),
)


def _fin_body(acc_ref, hprev_ref, dinv_ref, b_ref, wc_ref, bc_ref,
              psum_ref, logits_ref):
    i = pl.program_id(0)
    out3 = (acc_ref[0] + acc_ref[1] + hprev_ref[...]) * dinv_ref[...] + b_ref[...]

    @pl.when(i == 0)
    def _():
        psum_ref[...] = jnp.zeros_like(psum_ref)

    psum_ref[...] += jnp.sum(out3, axis=0, keepdims=True)
    pooled = psum_ref[...] * (1.0 / N)
    logits_ref[...] = _xwt(pooled, wc_ref[...]) + bc_ref[...]


_fin = pl.pallas_call(
    _fin_body,
    grid=(GRID,),
    in_specs=[
        pl.BlockSpec((NC, BLK, D), lambda i: (0, i, 0)),
        pl.BlockSpec((BLK, D), lambda i: (i, 0)),
        pl.BlockSpec((BLK, 1), lambda i: (i, 0)),
        pl.BlockSpec((1, D), lambda i: (0, 0)),
        pl.BlockSpec((D_OUT, D), lambda i: (0, 0)),
        pl.BlockSpec((1, D_OUT), lambda i: (0, 0)),
    ],
    out_specs=[
        pl.BlockSpec((1, D), lambda i: (0, 0)),
        pl.BlockSpec((1, D_OUT), lambda i: (0, 0)),
    ],
    out_shape=[
        jax.ShapeDtypeStruct((1, D), jnp.float32),
        jax.ShapeDtypeStruct((1, D_OUT), jnp.float32),
    ],
)


def _pad_edges(edge_index):
    npad = EP - E
    row = edge_index[0].astype(jnp.int32)
    col = edge_index[1].astype(jnp.int32)
    row_p = jnp.concatenate([row, jnp.zeros((npad,), jnp.int32)])
    col_p = jnp.concatenate(
        [col, N + (jnp.arange(npad, dtype=jnp.int32) % (NPAD - N))])
    return row_p.reshape(NW, NCHUNK, CH), col_p.reshape(NW, NCHUNK, CH)


def kernel(x, edge_index, W1, b1, W2, b2, W3, b3, Wc, bc):
    row, col = _pad_edges(edge_index)
    rowf = row.reshape(NCHUNKS_TOTAL, CH)
    colf = col.reshape(NCHUNKS_TOTAL, CH)
    ones_b = jnp.ones((CH, D), jnp.float32)
    zeros_acc = jnp.zeros((RSZ, D), jnp.float32)

    deg = _deg_kernel(col, ones_b, zeros_acc)
    dinv, h1p = _mm1(deg, x, W1)
    acc1 = _gs_kernel(h1p, rowf, colf, zeros_acc)
    h2p = _mid(acc1, h1p, dinv, b1.reshape(1, D), W2)
    acc2 = _gs_kernel(h2p, rowf, colf, zeros_acc)
    h3p = _mid(acc2, h2p, dinv, b2.reshape(1, D), W3)
    acc3 = _gs_kernel(h3p, rowf, colf, zeros_acc)
    _, logits = _fin(acc3, h3p, dinv, b3.reshape(1, D), Wc, bc.reshape(1, D_OUT))
    return logits


# speedup vs baseline: 1.2625x; 1.2625x over previous
import functools

import jax
import jax.numpy as jnp
from jax import lax
from jax.experimental import pallas as pl
from jax.experimental.pallas import tpu as pltpu
from jax.experimental.pallas import tpu_sc as plsc

N = 10000
E = 320000
D = 128
D_OUT = 64

NC = 2
NS = 16
NW = NC * NS
CH = 128
NCHUNK = 80
GRP = 40
NGRP = NCHUNK // GRP
EP = NW * NCHUNK * CH
NPAD = 10016
RSTRIDE = 624
RSZ = 656
DEGW = 16

_sc_mesh = plsc.VectorSubcoreMesh(core_axis_name="c", subcore_axis_name="s")


@functools.partial(
    pl.kernel,
    out_type=jax.ShapeDtypeStruct((NC, NPAD, D), jnp.float32),
    mesh=_sc_mesh,
    scratch_types=[
        pltpu.VMEM((NCHUNK, CH), jnp.int32),
        pltpu.VMEM((CH, D), jnp.float32),
        pltpu.VMEM_SHARED((NPAD, D), jnp.float32),
    ],
)
def _deg_kernel(col_hbm, ones_hbm, zeros_hbm, deg_out, idx_v, ones_v, table_s):
    c = lax.axis_index("c")
    s = lax.axis_index("s")
    wid = c * NS + s
    sl = pl.ds(s * RSTRIDE, RSZ)
    pltpu.sync_copy(col_hbm.at[wid], idx_v)
    pltpu.sync_copy(ones_hbm, ones_v)
    pltpu.sync_copy(zeros_hbm, table_s.at[sl])
    plsc.subcore_barrier()

    def body(j, carry):
        pltpu.sync_copy(ones_v, table_s.at[idx_v.at[j]], add=True)
        return carry

    lax.fori_loop(0, NCHUNK, body, 0)
    plsc.subcore_barrier()
    pltpu.sync_copy(table_s.at[sl], deg_out.at[c, sl])


@functools.partial(
    pl.kernel,
    out_type=jax.ShapeDtypeStruct((NC, NPAD, D), jnp.float32),
    mesh=_sc_mesh,
    scratch_types=[
        pltpu.VMEM((GRP, CH), jnp.int32),
        pltpu.VMEM((GRP, CH), jnp.int32),
        pltpu.VMEM((CH, D), jnp.float32),
        pltpu.VMEM((CH, D), jnp.float32),
        pltpu.VMEM_SHARED((NPAD, D), jnp.float32),
        pltpu.SemaphoreType.DMA,
        pltpu.SemaphoreType.DMA,
    ],
)
def _gs_kernel(hp_hbm, row_hbm, col_hbm, zeros_hbm, acc_out,
               row_v, col_v, bufa, bufb, acc_s, sema, semb):
    c = lax.axis_index("c")
    s = lax.axis_index("s")
    wid = c * NS + s
    sl = pl.ds(s * RSTRIDE, RSZ)
    pltpu.sync_copy(row_hbm.at[wid, pl.ds(0, GRP)], row_v)
    pltpu.sync_copy(col_hbm.at[wid, pl.ds(0, GRP)], col_v)
    pltpu.async_copy(hp_hbm.at[row_v.at[0]], bufa, sema)
    pltpu.sync_copy(zeros_hbm, acc_s.at[sl])
    plsc.subcore_barrier()

    def body(jj, carry2):
        j0 = jj * 2
        pltpu.async_copy(hp_hbm.at[row_v.at[j0 + 1]], bufb, semb)
        pltpu.make_async_copy(hp_hbm.at[row_v.at[j0]], bufa, sema).wait()
        pltpu.sync_copy(bufa, acc_s.at[col_v.at[j0]], add=True)

        @pl.when(jj < GRP // 2 - 1)
        def _():
            pltpu.async_copy(hp_hbm.at[row_v.at[j0 + 2]], bufa, sema)

        pltpu.make_async_copy(hp_hbm.at[row_v.at[j0 + 1]], bufb, semb).wait()
        pltpu.sync_copy(bufb, acc_s.at[col_v.at[j0 + 1]], add=True)
        return carry2

    for g in range(NGRP):
        lax.fori_loop(0, GRP // 2, body, 0)
        if g + 1 < NGRP:
            pltpu.sync_copy(row_hbm.at[wid, pl.ds((g + 1) * GRP, GRP)], row_v)
            pltpu.sync_copy(col_hbm.at[wid, pl.ds((g + 1) * GRP, GRP)], col_v)
            pltpu.async_copy(hp_hbm.at[row_v.at[0]], bufa, sema)

    plsc.subcore_barrier()
    pltpu.sync_copy(acc_s.at[sl], acc_out.at[c, sl])


BLK = 2000
GRID = N // BLK

_xwt = lambda a, b: lax.dot_general(
    a, b, (((1,), (1,)), ((), ())), preferred_element_type=jnp.float32)


def _mm1_body(deg_ref, x_ref, w_ref, dinv_ref, hp_ref):
    deg = deg_ref[0, :, 0:1] + deg_ref[1, :, 0:1] + 1.0
    dinv = lax.rsqrt(deg)
    dinv_ref[...] = dinv
    hp_ref[...] = _xwt(x_ref[...], w_ref[...]) * dinv


_mm1 = pl.pallas_call(
    _mm1_body,
    grid=(GRID,),
    in_specs=[
        pl.BlockSpec((NC, BLK, D), lambda i: (0, i, 0)),
        pl.BlockSpec((BLK, D), lambda i: (i, 0)),
        pl.BlockSpec((D, D), lambda i: (0, 0)),
    ],
    out_specs=[
        pl.BlockSpec((BLK, 1), lambda i: (i, 0)),
        pl.BlockSpec((BLK, D), lambda i: (i, 0)),
    ],
    out_shape=[
        jax.ShapeDtypeStruct((N, 1), jnp.float32),
        jax.ShapeDtypeStruct((N, D), jnp.float32),
    ],
)


def _mid_body(acc_ref, hprev_ref, dinv_ref, b_ref, w_ref, hp_ref):
    dinv = dinv_ref[...]
    pre = (acc_ref[0] + acc_ref[1] + hprev_ref[...]) * dinv + b_ref[...]
    xn = jnp.maximum(pre, 0.0)
    hp_ref[...] = _xwt(xn, w_ref[...]) * dinv


_mid = pl.pallas_call(
    _mid_body,
    grid=(GRID,),
    in_specs=[
        pl.BlockSpec((NC, BLK, D), lambda i: (0, i, 0)),
        pl.BlockSpec((BLK, D), lambda i: (i, 0)),
        pl.BlockSpec((BLK, 1), lambda i: (i, 0)),
        pl.BlockSpec((1, D), lambda i: (0, 0)),
        pl.BlockSpec((D, D), lambda i: (0, 0)),
    ],
    out_specs=pl.BlockSpec((BLK, D), lambda i: (i, 0)),
    out_shape=jax.ShapeDtypeStruct((N, D), jnp.float32),
)


def _fin_body(acc_ref, hprev_ref, dinv_ref, b_ref, wc_ref, bc_ref,
              psum_ref, logits_ref):
    i = pl.program_id(0)
    out3 = (acc_ref[0] + acc_ref[1] + hprev_ref[...]) * dinv_ref[...] + b_ref[...]

    @pl.when(i == 0)
    def _():
        psum_ref[...] = jnp.zeros_like(psum_ref)

    psum_ref[...] += jnp.sum(out3, axis=0, keepdims=True)
    pooled = psum_ref[...] * (1.0 / N)
    logits_ref[...] = _xwt(pooled, wc_ref[...]) + bc_ref[...]


_fin = pl.pallas_call(
    _fin_body,
    grid=(GRID,),
    in_specs=[
        pl.BlockSpec((NC, BLK, D), lambda i: (0, i, 0)),
        pl.BlockSpec((BLK, D), lambda i: (i, 0)),
        pl.BlockSpec((BLK, 1), lambda i: (i, 0)),
        pl.BlockSpec((1, D), lambda i: (0, 0)),
        pl.BlockSpec((D_OUT, D), lambda i: (0, 0)),
        pl.BlockSpec((1, D_OUT), lambda i: (0, 0)),
    ],
    out_specs=[
        pl.BlockSpec((1, D), lambda i: (0, 0)),
        pl.BlockSpec((1, D_OUT), lambda i: (0, 0)),
    ],
    out_shape=[
        jax.ShapeDtypeStruct((1, D), jnp.float32),
        jax.ShapeDtypeStruct((1, D_OUT), jnp.float32),
    ],
)


def _pad_edges(edge_index):
    npad = EP - E
    row = edge_index[0].astype(jnp.int32)
    col = edge_index[1].astype(jnp.int32)
    row_p = jnp.concatenate([row, jnp.zeros((npad,), jnp.int32)])
    col_p = jnp.concatenate(
        [col, N + (jnp.arange(npad, dtype=jnp.int32) % (NPAD - N))])
    return row_p.reshape(NW, NCHUNK, CH), col_p.reshape(NW, NCHUNK, CH)


def kernel(x, edge_index, W1, b1, W2, b2, W3, b3, Wc, bc):
    row, col = _pad_edges(edge_index)
    ones_b = jnp.ones((CH, D), jnp.float32)
    zeros_acc = jnp.zeros((RSZ, D), jnp.float32)

    deg = _deg_kernel(col, ones_b, zeros_acc)
    dinv, h1p = _mm1(deg, x, W1)
    acc1 = _gs_kernel(h1p, row, col, zeros_acc)
    h2p = _mid(acc1, h1p, dinv, b1.reshape(1, D), W2)
    acc2 = _gs_kernel(h2p, row, col, zeros_acc)
    h3p = _mid(acc2, h2p, dinv, b2.reshape(1, D), W3)
    acc3 = _gs_kernel(h3p, row, col, zeros_acc)
    _, logits = _fin(acc3, h3p, dinv, b3.reshape(1, D), Wc, bc.reshape(1, D_OUT))
    return logits
